# baseline (speedup 1.0000x reference)
import functools
import math

import jax
import jax.numpy as jnp
from jax import lax
from jax.experimental import pallas as pl
from jax.experimental.pallas import tpu as pltpu
from jax.experimental.pallas import tpu_sc as plsc

D_EMBED = 128
D_MODEL = 768
SEQ = 8192

_NC = 2
_NS = 16
_NW = _NC * _NS


def _sc_gather(table, idx):
    n = idx.shape[0]
    rows_per_w = n // _NW
    ch = 128
    n_ch = rows_per_w // ch
    idx3 = idx.reshape(_NW, n_ch, ch)

    mesh = plsc.VectorSubcoreMesh(core_axis_name="c", subcore_axis_name="s")

    @functools.partial(
        pl.kernel,
        out_type=jax.ShapeDtypeStruct((n, D_EMBED), jnp.float32),
        mesh=mesh,
        scratch_types=[
            pltpu.VMEM((n_ch, ch), jnp.int32),
            pltpu.VMEM((2, ch, D_EMBED), jnp.float32),
            pltpu.SemaphoreType.DMA,
            pltpu.SemaphoreType.DMA,
        ],
    )
    def gather_kernel(table_hbm, idx_hbm, out_hbm, idx_v, rows_v, sem0, sem1):
        wid = lax.axis_index("s") * _NC + lax.axis_index("c")
        base = wid * rows_per_w
        pltpu.sync_copy(idx_hbm.at[wid], idx_v)
        sems = [sem0, sem1]
        cps = [None, None]
        for c in range(n_ch):
            buf = c % 2
            cps[buf] = pltpu.async_copy(
                table_hbm.at[idx_v.at[c]], rows_v.at[buf], sems[buf]
            )
            if c > 0:
                pbuf = (c - 1) % 2
                cps[pbuf].wait()
                pltpu.sync_copy(
                    rows_v.at[pbuf], out_hbm.at[pl.ds(base + (c - 1) * ch, ch)]
                )
        lbuf = (n_ch - 1) % 2
        cps[lbuf].wait()
        pltpu.sync_copy(
            rows_v.at[lbuf], out_hbm.at[pl.ds(base + (n_ch - 1) * ch, ch)]
        )

    return gather_kernel(table, idx3)


def _tc_dense(emb, wt, b, gamma, beta, n_total, row0, prev=None):
    n = emb.shape[0]
    tile = 2048
    sub = 512
    grid = n // tile
    blk0 = row0 // tile
    scale = math.sqrt(float(D_MODEL))

    def body(e_ref, wt_ref, b_ref, g_ref, bt_ref, *rest):
        if prev is None:
            o_ref, s_ref, c_ref = rest
        else:
            _, o_ref, s_ref, c_ref = rest
        li = pl.program_id(0)
        i = li + blk0
        col = lax.broadcasted_iota(jnp.int32, (1, D_MODEL), 1)
        odd = col % 2
        ceven = (col - odd).astype(jnp.float32)
        freq = jnp.exp(-ceven / float(D_MODEL) * 4.0 * math.log(10.0))

        @pl.when(li == 0)
        def _():
            r = lax.broadcasted_iota(jnp.int32, (sub, 1), 0).astype(jnp.float32)
            ang = r * freq
            s_ref[...] = jnp.sin(ang)
            c_ref[...] = jnp.cos(ang)

        h = jnp.dot(e_ref[...], wt_ref[...], preferred_element_type=jnp.float32)
        h = (h + b_ref[...]) * scale

        is_odd = odd == 1
        pes = []
        for k in range(tile // sub):
            pos0 = jnp.float32((i * tile + k * sub) % SEQ)
            ang0 = pos0 * freq
            s0 = jnp.sin(ang0)
            c0 = jnp.cos(ang0)
            sa = jnp.where(is_odd, c0, s0)
            ca = jnp.where(is_odd, -s0, c0)
            pes.append(sa * c_ref[...] + ca * s_ref[...])
        h = h + jnp.concatenate(pes, axis=0)

        m = jnp.mean(h, axis=1, keepdims=True)
        d = h - m
        v = jnp.mean(d * d, axis=1, keepdims=True)
        o_ref[...] = d * lax.rsqrt(v + 1e-5) * g_ref[...] + bt_ref[...]

    in_specs = [
        pl.BlockSpec((tile, D_EMBED), lambda i: (i, 0)),
        pl.BlockSpec((D_EMBED, D_MODEL), lambda i: (0, 0)),
        pl.BlockSpec((1, D_MODEL), lambda i: (0, 0)),
        pl.BlockSpec((1, D_MODEL), lambda i: (0, 0)),
        pl.BlockSpec((1, D_MODEL), lambda i: (0, 0)),
    ]
    args = [emb, wt, b, gamma, beta]
    kwargs = {}
    if prev is not None:
        in_specs.append(pl.BlockSpec(memory_space=pl.ANY))
        args.append(prev)
        kwargs["input_output_aliases"] = {5: 0}

    def out_map(i, _blk0=blk0):
        return (i + _blk0, 0)

    return pl.pallas_call(
        body,
        grid=(grid,),
        in_specs=in_specs,
        out_specs=pl.BlockSpec((tile, D_MODEL), out_map),
        out_shape=jax.ShapeDtypeStruct((n_total, D_MODEL), jnp.float32),
        scratch_shapes=[
            pltpu.VMEM((sub, D_MODEL), jnp.float32),
            pltpu.VMEM((sub, D_MODEL), jnp.float32),
        ],
        **kwargs,
    )(*args)


def kernel(x, table, W, b, gamma, beta):
    bsz, seq = x.shape
    idx = x.reshape(-1).astype(jnp.int32)
    n = idx.shape[0]
    half = n // 2
    wt = W.T
    b2 = b.reshape(1, D_MODEL)
    g2 = gamma.reshape(1, D_MODEL)
    bt2 = beta.reshape(1, D_MODEL)

    emb1 = _sc_gather(table, idx[:half])
    emb2 = _sc_gather(table, idx[half:])
    out = _tc_dense(emb1, wt, b2, g2, bt2, n, 0)
    out = _tc_dense(emb2, wt, b2, g2, bt2, n, half, prev=out)
    return out.reshape(bsz, seq, D_MODEL)


# speedup vs baseline: 1.0233x; 1.0233x over previous
import functools
import math

import jax
import jax.numpy as jnp
from jax import lax
from jax.experimental import pallas as pl
from jax.experimental.pallas import tpu as pltpu
from jax.experimental.pallas import tpu_sc as plsc

D_EMBED = 128
D_MODEL = 768
SEQ = 8192

_NC = 2
_NS = 16
_NW = _NC * _NS


def _sc_gather(table, idx):
    n = idx.shape[0]
    rows_per_w = n // _NW
    ch = 128
    n_ch = rows_per_w // ch
    idx3 = idx.reshape(_NW, n_ch, ch)

    mesh = plsc.VectorSubcoreMesh(core_axis_name="c", subcore_axis_name="s")

    @functools.partial(
        pl.kernel,
        out_type=jax.ShapeDtypeStruct((n, D_EMBED), jnp.float32),
        mesh=mesh,
        scratch_types=[
            pltpu.VMEM((n_ch, ch), jnp.int32),
            pltpu.VMEM((2, ch, D_EMBED), jnp.float32),
            pltpu.SemaphoreType.DMA,
            pltpu.SemaphoreType.DMA,
        ],
    )
    def gather_kernel(table_hbm, idx_hbm, out_hbm, idx_v, rows_v, sem0, sem1):
        wid = lax.axis_index("s") * _NC + lax.axis_index("c")
        base = wid * rows_per_w
        pltpu.sync_copy(idx_hbm.at[wid], idx_v)
        sems = [sem0, sem1]
        cps = [None, None]
        for c in range(n_ch):
            buf = c % 2
            cps[buf] = pltpu.async_copy(
                table_hbm.at[idx_v.at[c]], rows_v.at[buf], sems[buf]
            )
            if c > 0:
                pbuf = (c - 1) % 2
                cps[pbuf].wait()
                pltpu.sync_copy(
                    rows_v.at[pbuf], out_hbm.at[pl.ds(base + (c - 1) * ch, ch)]
                )
        lbuf = (n_ch - 1) % 2
        cps[lbuf].wait()
        pltpu.sync_copy(
            rows_v.at[lbuf], out_hbm.at[pl.ds(base + (n_ch - 1) * ch, ch)]
        )

    return gather_kernel(table, idx3)


def _tc_dense(emb, wt, b, gamma, beta, n_total, row0, prev=None):
    n = emb.shape[0]
    tile = 4096
    sub = 512
    grid = n // tile
    blk0 = row0 // tile
    scale = math.sqrt(float(D_MODEL))

    def body(e_ref, wt_ref, b_ref, g_ref, bt_ref, *rest):
        if prev is None:
            o_ref, s_ref, c_ref = rest
        else:
            _, o_ref, s_ref, c_ref = rest
        li = pl.program_id(0)
        i = li + blk0
        col = lax.broadcasted_iota(jnp.int32, (1, D_MODEL), 1)
        odd = col % 2
        ceven = (col - odd).astype(jnp.float32)
        freq = jnp.exp(-ceven / float(D_MODEL) * 4.0 * math.log(10.0))

        @pl.when(li == 0)
        def _():
            r = lax.broadcasted_iota(jnp.int32, (sub, 1), 0).astype(jnp.float32)
            ang = r * freq
            s_ref[...] = jnp.sin(ang)
            c_ref[...] = jnp.cos(ang)

        h = jnp.dot(e_ref[...], wt_ref[...], preferred_element_type=jnp.float32)
        h = (h + b_ref[...]) * scale

        is_odd = odd == 1
        pes = []
        for k in range(tile // sub):
            pos0 = jnp.float32((i * tile + k * sub) % SEQ)
            ang0 = pos0 * freq
            s0 = jnp.sin(ang0)
            c0 = jnp.cos(ang0)
            sa = jnp.where(is_odd, c0, s0)
            ca = jnp.where(is_odd, -s0, c0)
            pes.append(sa * c_ref[...] + ca * s_ref[...])
        h = h + jnp.concatenate(pes, axis=0)

        m = jnp.mean(h, axis=1, keepdims=True)
        d = h - m
        v = jnp.mean(d * d, axis=1, keepdims=True)
        o_ref[...] = d * lax.rsqrt(v + 1e-5) * g_ref[...] + bt_ref[...]

    in_specs = [
        pl.BlockSpec((tile, D_EMBED), lambda i: (i, 0)),
        pl.BlockSpec((D_EMBED, D_MODEL), lambda i: (0, 0)),
        pl.BlockSpec((1, D_MODEL), lambda i: (0, 0)),
        pl.BlockSpec((1, D_MODEL), lambda i: (0, 0)),
        pl.BlockSpec((1, D_MODEL), lambda i: (0, 0)),
    ]
    args = [emb, wt, b, gamma, beta]
    kwargs = {}
    if prev is not None:
        in_specs.append(pl.BlockSpec(memory_space=pl.ANY))
        args.append(prev)
        kwargs["input_output_aliases"] = {5: 0}

    def out_map(i, _blk0=blk0):
        return (i + _blk0, 0)

    return pl.pallas_call(
        body,
        grid=(grid,),
        in_specs=in_specs,
        out_specs=pl.BlockSpec((tile, D_MODEL), out_map),
        out_shape=jax.ShapeDtypeStruct((n_total, D_MODEL), jnp.float32),
        scratch_shapes=[
            pltpu.VMEM((sub, D_MODEL), jnp.float32),
            pltpu.VMEM((sub, D_MODEL), jnp.float32),
        ],
        **kwargs,
    )(*args)


def kernel(x, table, W, b, gamma, beta):
    bsz, seq = x.shape
    idx = x.reshape(-1).astype(jnp.int32)
    n = idx.shape[0]
    half = n // 2
    wt = W.T
    b2 = b.reshape(1, D_MODEL)
    g2 = gamma.reshape(1, D_MODEL)
    bt2 = beta.reshape(1, D_MODEL)

    emb = _sc_gather(table, idx)
    out = _tc_dense(emb, wt, b2, g2, bt2, n, 0)
    return out.reshape(bsz, seq, D_MODEL)


# speedup vs baseline: 1.0494x; 1.0256x over previous
import functools
import math

import jax
import jax.numpy as jnp
from jax import lax
from jax.experimental import pallas as pl
from jax.experimental.pallas import tpu as pltpu
from jax.experimental.pallas import tpu_sc as plsc

D_EMBED = 128
D_MODEL = 768
SEQ = 8192

_NC = 2
_NS = 16
_NW = _NC * _NS


def _sc_gather(table, idx):
    n = idx.shape[0]
    rows_per_w = n // _NW
    ch = 128
    n_ch = rows_per_w // ch
    idx3 = idx.reshape(_NW, n_ch, ch)

    mesh = plsc.VectorSubcoreMesh(core_axis_name="c", subcore_axis_name="s")

    @functools.partial(
        pl.kernel,
        out_type=jax.ShapeDtypeStruct((n, D_EMBED), jnp.float32),
        mesh=mesh,
        scratch_types=[
            pltpu.VMEM((n_ch, ch), jnp.int32),
            pltpu.VMEM((4, ch, D_EMBED), jnp.float32),
            pltpu.SemaphoreType.DMA,
            pltpu.SemaphoreType.DMA,
            pltpu.SemaphoreType.DMA,
            pltpu.SemaphoreType.DMA,
            pltpu.SemaphoreType.DMA,
            pltpu.SemaphoreType.DMA,
            pltpu.SemaphoreType.DMA,
            pltpu.SemaphoreType.DMA,
        ],
    )
    def gather_kernel(table_hbm, idx_hbm, out_hbm, idx_v, rows_v, *sems):
        g_sems, s_sems = sems[:4], sems[4:]
        wid = lax.axis_index("s") * _NC + lax.axis_index("c")
        base = wid * rows_per_w
        pltpu.sync_copy(idx_hbm.at[wid], idx_v)
        g_cp = [None] * 4
        s_cp = [None] * 4
        for c in range(n_ch):
            buf = c % 4
            if c >= 4:
                s_cp[buf].wait()
            g_cp[buf] = pltpu.async_copy(
                table_hbm.at[idx_v.at[c]], rows_v.at[buf], g_sems[buf]
            )
            d = c - 3
            if d >= 0:
                dbuf = d % 4
                g_cp[dbuf].wait()
                s_cp[dbuf] = pltpu.async_copy(
                    rows_v.at[dbuf],
                    out_hbm.at[pl.ds(base + d * ch, ch)],
                    s_sems[dbuf],
                )
        for d in range(n_ch - 3, n_ch):
            dbuf = d % 4
            g_cp[dbuf].wait()
            s_cp[dbuf] = pltpu.async_copy(
                rows_v.at[dbuf],
                out_hbm.at[pl.ds(base + d * ch, ch)],
                s_sems[dbuf],
            )
        for d in range(n_ch - 4, n_ch):
            s_cp[d % 4].wait()

    return gather_kernel(table, idx3)


def _tc_dense(emb, wt, b, gamma, beta, n_total, row0, prev=None):
    n = emb.shape[0]
    tile = 2048
    sub = 512
    grid = n // tile
    blk0 = row0 // tile
    scale = math.sqrt(float(D_MODEL))

    def body(e_ref, wt_ref, b_ref, g_ref, bt_ref, *rest):
        if prev is None:
            o_ref, s_ref, c_ref = rest
        else:
            _, o_ref, s_ref, c_ref = rest
        li = pl.program_id(0)
        i = li + blk0
        col = lax.broadcasted_iota(jnp.int32, (1, D_MODEL), 1)
        odd = col % 2
        ceven = (col - odd).astype(jnp.float32)
        freq = jnp.exp(-ceven / float(D_MODEL) * 4.0 * math.log(10.0))

        @pl.when(li == 0)
        def _():
            r = lax.broadcasted_iota(jnp.int32, (sub, 1), 0).astype(jnp.float32)
            ang = r * freq
            s_ref[...] = jnp.sin(ang)
            c_ref[...] = jnp.cos(ang)

        h = jnp.dot(e_ref[...], wt_ref[...], preferred_element_type=jnp.float32)
        h = (h + b_ref[...]) * scale

        is_odd = odd == 1
        pes = []
        for k in range(tile // sub):
            pos0 = jnp.float32((i * tile + k * sub) % SEQ)
            ang0 = pos0 * freq
            s0 = jnp.sin(ang0)
            c0 = jnp.cos(ang0)
            sa = jnp.where(is_odd, c0, s0)
            ca = jnp.where(is_odd, -s0, c0)
            pes.append(sa * c_ref[...] + ca * s_ref[...])
        h = h + jnp.concatenate(pes, axis=0)

        m = jnp.mean(h, axis=1, keepdims=True)
        d = h - m
        v = jnp.mean(d * d, axis=1, keepdims=True)
        o_ref[...] = d * lax.rsqrt(v + 1e-5) * g_ref[...] + bt_ref[...]

    in_specs = [
        pl.BlockSpec((tile, D_EMBED), lambda i: (i, 0)),
        pl.BlockSpec((D_EMBED, D_MODEL), lambda i: (0, 0)),
        pl.BlockSpec((1, D_MODEL), lambda i: (0, 0)),
        pl.BlockSpec((1, D_MODEL), lambda i: (0, 0)),
        pl.BlockSpec((1, D_MODEL), lambda i: (0, 0)),
    ]
    args = [emb, wt, b, gamma, beta]
    kwargs = {}
    if prev is not None:
        in_specs.append(pl.BlockSpec(memory_space=pl.ANY))
        args.append(prev)
        kwargs["input_output_aliases"] = {5: 0}

    def out_map(i, _blk0=blk0):
        return (i + _blk0, 0)

    return pl.pallas_call(
        body,
        grid=(grid,),
        in_specs=in_specs,
        out_specs=pl.BlockSpec((tile, D_MODEL), out_map),
        out_shape=jax.ShapeDtypeStruct((n_total, D_MODEL), jnp.float32),
        scratch_shapes=[
            pltpu.VMEM((sub, D_MODEL), jnp.float32),
            pltpu.VMEM((sub, D_MODEL), jnp.float32),
        ],
        **kwargs,
    )(*args)


def kernel(x, table, W, b, gamma, beta):
    bsz, seq = x.shape
    idx = x.reshape(-1).astype(jnp.int32)
    n = idx.shape[0]
    half = n // 2
    wt = W.T
    b2 = b.reshape(1, D_MODEL)
    g2 = gamma.reshape(1, D_MODEL)
    bt2 = beta.reshape(1, D_MODEL)

    emb = _sc_gather(table, idx)
    out = _tc_dense(emb, wt, b2, g2, bt2, n, 0)
    return out.reshape(bsz, seq, D_MODEL)
